# baseline (speedup 1.0000x reference)
import functools

import jax
import jax.numpy as jnp
from jax import lax
from jax.experimental import pallas as pl
from jax.experimental.pallas import tpu as pltpu
from jax.experimental.pallas import tpu_sc as plsc

PRE_NMS_THRESH = 0.01
PRE_NMS_TOP_N = 1000
FPN_POST_NMS_TOP_N = 100

CAP = 2048
LANES = 128
SCORE_CHUNK = 1152
COUNT_CHUNK = 1296
NW = 32
HI_BITS = 0x3F800001
DUMP = 1024


def _score_body(cls_ref, ctr_ref, s_ref, tau_ref, cnt_ref, sub_ref):
    C, HW = cls_ref.shape[1], cls_ref.shape[2]
    pad = jnp.full((SCORE_CHUNK, LANES - 80), -1.0, jnp.float32)
    sub_rows = SCORE_CHUNK // 16

    def score_chunk(j, _):
        c = cls_ref[0, :, pl.ds(j * SCORE_CHUNK, SCORE_CHUNK)]
        t = ctr_ref[0, :, pl.ds(j * SCORE_CHUNK, SCORE_CHUNK)]
        sT = jnp.transpose(jax.nn.sigmoid(c))
        stT = jnp.transpose(jax.nn.sigmoid(t))
        msk = jnp.where(sT > PRE_NMS_THRESH, sT * stT, -1.0)
        full = jnp.concatenate([msk, pad], axis=1)
        s_ref[0, pl.ds(j * SCORE_CHUNK, SCORE_CHUNK), :] = full
        sub_ref[pl.ds(j * sub_rows, sub_rows), :] = full[0:sub_rows, :]
        return 0
    lax.fori_loop(0, HW // SCORE_CHUNK, score_chunk, 0)

    def count_ge(tau_bits):
        def cbody(i, acc):
            blk = s_ref[0, pl.ds(i * COUNT_CHUNK, COUNT_CHUNK), :]
            bits = lax.bitcast_convert_type(blk, jnp.int32)
            return acc + jnp.sum((bits >= tau_bits).astype(jnp.int32))
        return lax.fori_loop(0, HW // COUNT_CHUNK, cbody, jnp.int32(0))

    def count_sub(tau_bits):
        bits = lax.bitcast_convert_type(sub_ref[...], jnp.int32)
        return jnp.sum((bits >= tau_bits).astype(jnp.int32))

    SUB_LO, SUB_HI = 96, 120

    def sbis_cond(st):
        lo, hi, cnt, it = st
        bad = (cnt < SUB_LO) | (cnt > SUB_HI)
        return bad & (it < 24) & (lo + 1 < hi)

    def sbis_body(st):
        lo, hi, cnt, it = st
        mid = (lo + hi) // 2
        c = count_sub(mid)
        ok = c >= SUB_LO
        return (jnp.where(ok, mid, lo), jnp.where(ok, hi, mid),
                jnp.where(ok, c, cnt), it + 1)

    t1 = jnp.int32(0x3F700000)

    def bis_cond(st):
        lo, hi, cnt, it = st
        bad = (cnt < PRE_NMS_TOP_N) | (cnt > CAP)
        return bad & (it < 34) & (lo + 1 < hi)

    def bis_body(st):
        lo, hi, cnt, it = st
        mid = (lo + hi) // 2
        c = count_ge(mid)
        ok = c >= PRE_NMS_TOP_N
        return (jnp.where(ok, mid, lo), jnp.where(ok, hi, mid),
                jnp.where(ok, c, cnt), it + 1)

    tau_ref[...] = lax.broadcast(t1, (1, 1, 1))
    tau = t1
    if True:
        rows_w0 = HW // NW
        lane0 = lax.broadcasted_iota(jnp.int32, (1, NW), 1)

        def wcount0(w, cvec):
            blk = s_ref[0, pl.ds(w * rows_w0, rows_w0), :]
            bits = lax.bitcast_convert_type(blk, jnp.int32)
            cw = jnp.sum((bits >= tau).astype(jnp.int32))
            return jnp.where(lane0 == w, cw, cvec)

        cvec0 = lax.fori_loop(0, NW, wcount0, jnp.zeros((1, NW), jnp.int32))
        cnt_ref[...] = cvec0.reshape(1, 1, NW)
        return

    c1 = count_ge(t1)
    lo0 = jnp.where(c1 >= PRE_NMS_TOP_N, t1, jnp.int32(0))
    hi0 = jnp.where(c1 >= PRE_NMS_TOP_N, jnp.int32(HI_BITS), t1)
    cnt0 = jnp.where(c1 >= PRE_NMS_TOP_N, c1, jnp.int32(0x7FFFFFF0))
    tau, _, _, _ = lax.while_loop(
        bis_cond, bis_body, (lo0, hi0, cnt0, jnp.int32(0)))
    tau_ref[...] = lax.broadcast(tau, (1, 1, 1))

    rows_w = HW // NW
    lane = lax.broadcasted_iota(jnp.int32, (1, NW), 1)

    def wcount(w, cvec):
        blk = s_ref[0, pl.ds(w * rows_w, rows_w), :]
        bits = lax.bitcast_convert_type(blk, jnp.int32)
        cw = jnp.sum((bits >= tau).astype(jnp.int32))
        return jnp.where(lane == w, cw, cvec)

    cvec = lax.fori_loop(0, NW, wcount, jnp.zeros((1, NW), jnp.int32))
    cnt_ref[...] = cvec.reshape(1, 1, NW)


def _tc_select(cls_nat, ctr_nat):
    N, C, HW = cls_nat.shape
    return pl.pallas_call(
        _score_body,
        grid=(N,),
        in_specs=[
            pl.BlockSpec((1, C, HW), lambda n: (n, 0, 0)),
            pl.BlockSpec((1, 1, HW), lambda n: (n, 0, 0)),
        ],
        out_specs=[
            pl.BlockSpec((1, HW, LANES), lambda n: (n, 0, 0)),
            pl.BlockSpec((1, 1, 1), lambda n: (n, 0, 0)),
            pl.BlockSpec((1, 1, NW), lambda n: (n, 0, 0)),
        ],
        out_shape=[
            jax.ShapeDtypeStruct((N, HW, LANES), jnp.float32),
            jax.ShapeDtypeStruct((N, 1, 1), jnp.int32),
            jax.ShapeDtypeStruct((N, 1, NW), jnp.int32),
        ],
        scratch_shapes=[pltpu.VMEM((HW // 16, LANES), jnp.float32)],
    )(cls_nat, ctr_nat)


def _make_sc_compact(N, total, chunk):
    nvec = chunk // 16
    mesh = plsc.VectorSubcoreMesh(core_axis_name="c", subcore_axis_name="s")
    out_len = N * CAP + DUMP

    @functools.partial(
        pl.kernel, mesh=mesh,
        compiler_params=pltpu.CompilerParams(needs_layout_passes=False),
        out_type=[
            jax.ShapeDtypeStruct((out_len,), jnp.float32),
            jax.ShapeDtypeStruct((out_len,), jnp.int32),
        ],
        scratch_types=[
            pltpu.VMEM((chunk,), jnp.float32),
            pltpu.VMEM((16,), jnp.float32),
            pltpu.VMEM((16,), jnp.int32),
            pltpu.VMEM((CAP + 16,), jnp.float32),
            pltpu.VMEM((CAP + 16,), jnp.int32),
        ],
    )
    def sc_compact(scores_hbm, tau_hbm, base_hbm, outv_hbm, outi_hbm,
                   chunk_v, tau_v, base_v, valbuf, idxbuf):
        cid = lax.axis_index("c")
        sid = lax.axis_index("s")
        wid = sid * 2 + cid
        iota16 = lax.iota(jnp.int32, 16)
        one16 = jnp.ones((16,), jnp.int32)
        zero16 = jnp.zeros((16,), jnp.int32)

        for n in range(N):
            pltpu.sync_copy(
                scores_hbm.at[pl.ds(n * total + wid * chunk, chunk)], chunk_v)
            pltpu.sync_copy(tau_hbm.at[n], tau_v)
            pltpu.sync_copy(base_hbm.at[n, wid], base_v)
            tau16 = tau_v[...]
            base16 = base_v[...]

            def scatter_vec(i, ptrv):
                vals = chunk_v[pl.ds(i * 16, 16)]
                m = vals >= tau16
                mi = jnp.where(m, one16, zero16)
                incl = plsc.cumsum(mi)
                slots = ptrv + incl - mi
                ok = m & (slots < CAP + 16)
                fidx = wid * chunk + i * 16 + iota16
                plsc.store_scatter(valbuf, [slots], vals, mask=ok)
                plsc.store_scatter(idxbuf, [slots], fidx, mask=ok)
                return ptrv + plsc.all_reduce_population_count(m)

            def bbody(b, ptrv):
                i0 = b * 8
                mx = chunk_v[pl.ds(i0 * 16, 16)]
                for k in range(1, 8):
                    mx = jnp.maximum(mx, chunk_v[pl.ds((i0 + k) * 16, 16)])
                pc = plsc.all_reduce_population_count(mx >= tau16)

                def hitpath(p):
                    for k in range(8):
                        p = scatter_vec(i0 + k, p)
                    return p

                return lax.cond(pc[0] > 0, hitpath, lambda p: p, ptrv)

            cwv = lax.fori_loop(0, nvec // 8, bbody,
                                jnp.zeros((16,), jnp.int32))

            dump16 = jnp.int32(N * CAP) + wid * 16 + iota16

            def wbody(k, _):
                valid = (k * 16 + iota16) < cwv
                tgt = base16 + k * 16 + iota16
                safe = tgt < (n + 1) * CAP
                gidx = jnp.where(valid & safe, tgt, dump16)
                pltpu.sync_copy(valbuf.at[pl.ds(k * 16, 16)],
                                outv_hbm.at[gidx])
                pltpu.sync_copy(idxbuf.at[pl.ds(k * 16, 16)],
                                outi_hbm.at[gidx])
                return 0

            cw = cwv[0]
            lax.fori_loop(0, (cw + 15) >> 4, wbody, 0)

    return sc_compact


def kernel(locations, box_cls, box_regression, centerness, image_sizes):
    N, C, H, W = box_cls.shape
    HW = H * W
    total = HW * LANES
    chunk = total // NW
    cls_nat = box_cls.reshape(N, C, HW)
    ctr_nat = centerness.reshape(N, 1, HW)

    scores, tau, counts = _tc_select(cls_nat, ctr_nat)

    counts = counts.reshape(N, NW)
    offs = jnp.cumsum(counts, axis=1) - counts
    base = (jnp.arange(N, dtype=jnp.int32)[:, None] * CAP + offs)
    base16 = jnp.broadcast_to(base[:, :, None], (N, NW, 16)).astype(jnp.int32)
    tau_f = lax.bitcast_convert_type(tau.reshape(N, 1), jnp.float32)
    tau16 = jnp.broadcast_to(tau_f, (N, 16))
    c2 = counts.sum(axis=1)

    sc_compact = _make_sc_compact(N, total, chunk)
    outv_flat, outi_flat = sc_compact(scores.reshape(-1), tau16, base16)

    valid = jnp.arange(CAP, dtype=jnp.int32)[None, :] < c2[:, None]
    vals = jnp.where(valid, outv_flat[:N * CAP].reshape(N, CAP), -2.0)
    idxp = jnp.where(valid, outi_flat[:N * CAP].reshape(N, CAP), 0)
    idxl = (idxp >> 7) * C + (idxp & (LANES - 1))

    top_vals, tp = lax.top_k(vals, PRE_NMS_TOP_N)
    top_idx = jnp.take_along_axis(idxl, tp, axis=1)

    reg = jnp.transpose(box_regression, (0, 2, 3, 1)).reshape(N, HW, 4)
    loc_idx = top_idx // C
    labels = (top_idx % C) + 1
    per_reg = jnp.take_along_axis(reg, loc_idx[:, :, None], axis=1)
    per_loc = locations[loc_idx]
    x1 = per_loc[..., 0] - per_reg[..., 0]
    y1 = per_loc[..., 1] - per_reg[..., 1]
    x2 = per_loc[..., 0] + per_reg[..., 2]
    y2 = per_loc[..., 1] + per_reg[..., 3]
    w = jnp.maximum(image_sizes[:, 1], 2).astype(jnp.float32)[:, None]
    h = jnp.maximum(image_sizes[:, 0], 2).astype(jnp.float32)[:, None]
    x1 = jnp.clip(x1, 0.0, w - 1.0)
    x2 = jnp.clip(x2, 0.0, w - 1.0)
    y1 = jnp.clip(y1, 0.0, h - 1.0)
    y2 = jnp.clip(y2, 0.0, h - 1.0)
    ws = x2 - x1 + 1.0
    hs = y2 - y1 + 1.0
    keep = (ws >= 0) & (hs >= 0) & (top_vals > 0.0)
    final_scores = jnp.where(keep, top_vals, -1.0)
    fin_vals, fin_idx = lax.top_k(final_scores, FPN_POST_NMS_TOP_N)
    boxes = jnp.stack([x1, y1, x2, y2], axis=-1)
    fin_boxes = jnp.take_along_axis(boxes, fin_idx[:, :, None], axis=1)
    fin_labels = jnp.take_along_axis(labels, fin_idx, axis=1).astype(jnp.float32)
    out = jnp.concatenate([fin_boxes, fin_vals[:, :, None], fin_labels[:, :, None]], axis=-1)
    return out


# speedup vs baseline: 1.4737x; 1.4737x over previous
import functools

import jax
import jax.numpy as jnp
from jax import lax
from jax.experimental import pallas as pl
from jax.experimental.pallas import tpu as pltpu
from jax.experimental.pallas import tpu_sc as plsc

PRE_NMS_THRESH = 0.01
PRE_NMS_TOP_N = 1000
FPN_POST_NMS_TOP_N = 100

CAP = 2048
LANES = 128
SCORE_CHUNK = 1152
COUNT_CHUNK = 1296
NW = 32
HI_BITS = 0x3F800001
DUMP = 1024


def _score_body(cls_ref, ctr_ref, s_ref, tau_ref, cnt_ref, sub_ref):
    C, HW = cls_ref.shape[1], cls_ref.shape[2]
    pad = jnp.full((SCORE_CHUNK, LANES - 80), -1.0, jnp.float32)
    sub_rows = SCORE_CHUNK // 16

    def score_chunk(j, _):
        c = cls_ref[0, :, pl.ds(j * SCORE_CHUNK, SCORE_CHUNK)]
        t = ctr_ref[0, :, pl.ds(j * SCORE_CHUNK, SCORE_CHUNK)]
        sT = jnp.transpose(jax.nn.sigmoid(c))
        stT = jnp.transpose(jax.nn.sigmoid(t))
        msk = jnp.where(sT > PRE_NMS_THRESH, sT * stT, -1.0)
        full = jnp.concatenate([msk, pad], axis=1)
        s_ref[0, pl.ds(j * SCORE_CHUNK, SCORE_CHUNK), :] = full
        sub_ref[pl.ds(j * sub_rows, sub_rows), :] = full[0:sub_rows, :]
        return 0
    lax.fori_loop(0, HW // SCORE_CHUNK, score_chunk, 0)

    def count_ge(tau_bits):
        def cbody(i, acc):
            blk = s_ref[0, pl.ds(i * COUNT_CHUNK, COUNT_CHUNK), :]
            bits = lax.bitcast_convert_type(blk, jnp.int32)
            return acc + jnp.sum((bits >= tau_bits).astype(jnp.int32))
        return lax.fori_loop(0, HW // COUNT_CHUNK, cbody, jnp.int32(0))

    def count_sub(tau_bits):
        bits = lax.bitcast_convert_type(sub_ref[...], jnp.int32)
        return jnp.sum((bits >= tau_bits).astype(jnp.int32))

    SUB_LO, SUB_HI = 88, 126

    def sbis_cond(st):
        lo, hi, cnt, it = st
        bad = (cnt < SUB_LO) | (cnt > SUB_HI)
        return bad & (it < 24) & (lo + 1 < hi)

    def sbis_body(st):
        lo, hi, cnt, it = st
        mid = (lo + hi) // 2
        c = count_sub(mid)
        ok = c >= SUB_LO
        return (jnp.where(ok, mid, lo), jnp.where(ok, hi, mid),
                jnp.where(ok, c, cnt), it + 1)

    t1, _, _, _ = lax.while_loop(
        sbis_cond, sbis_body,
        (jnp.int32(0), jnp.int32(HI_BITS), count_sub(jnp.int32(0)),
         jnp.int32(0)))

    rows_w = HW // NW
    lane = lax.broadcasted_iota(jnp.int32, (1, NW), 1)

    def wcounts(tau_bits):
        def wcount(w, cvec):
            blk = s_ref[0, pl.ds(w * rows_w, rows_w), :]
            bits = lax.bitcast_convert_type(blk, jnp.int32)
            cw = jnp.sum((bits >= tau_bits).astype(jnp.int32))
            return jnp.where(lane == w, cw, cvec)
        return lax.fori_loop(0, NW, wcount, jnp.zeros((1, NW), jnp.int32))

    def bis_cond(st):
        lo, hi, cnt, it = st
        bad = (cnt < PRE_NMS_TOP_N) | (cnt > CAP)
        return bad & (it < 34) & (lo + 1 < hi)

    def bis_body(st):
        lo, hi, cnt, it = st
        mid = (lo + hi) // 2
        c = count_ge(mid)
        ok = c >= PRE_NMS_TOP_N
        return (jnp.where(ok, mid, lo), jnp.where(ok, hi, mid),
                jnp.where(ok, c, cnt), it + 1)

    cvec1 = wcounts(t1)
    c1 = jnp.sum(cvec1)
    lo0 = jnp.where(c1 >= PRE_NMS_TOP_N, t1, jnp.int32(0))
    hi0 = jnp.where(c1 >= PRE_NMS_TOP_N, jnp.int32(HI_BITS), t1)
    cnt0 = jnp.where(c1 >= PRE_NMS_TOP_N, c1, jnp.int32(0x7FFFFFF0))
    tau, _, _, _ = lax.while_loop(
        bis_cond, bis_body, (lo0, hi0, cnt0, jnp.int32(0)))
    tau_ref[...] = lax.broadcast(tau, (1, 1, 1))

    cvec = lax.cond(tau == t1, lambda: cvec1, lambda: wcounts(tau))
    cnt_ref[...] = cvec.reshape(1, 1, NW)


def _tc_select(cls_nat, ctr_nat):
    N, C, HW = cls_nat.shape
    return pl.pallas_call(
        _score_body,
        grid=(N,),
        in_specs=[
            pl.BlockSpec((1, C, HW), lambda n: (n, 0, 0)),
            pl.BlockSpec((1, 1, HW), lambda n: (n, 0, 0)),
        ],
        out_specs=[
            pl.BlockSpec((1, HW, LANES), lambda n: (n, 0, 0)),
            pl.BlockSpec((1, 1, 1), lambda n: (n, 0, 0)),
            pl.BlockSpec((1, 1, NW), lambda n: (n, 0, 0)),
        ],
        out_shape=[
            jax.ShapeDtypeStruct((N, HW, LANES), jnp.float32),
            jax.ShapeDtypeStruct((N, 1, 1), jnp.int32),
            jax.ShapeDtypeStruct((N, 1, NW), jnp.int32),
        ],
        scratch_shapes=[pltpu.VMEM((HW // 16, LANES), jnp.float32)],
    )(cls_nat, ctr_nat)


def _make_sc_compact(N, total, chunk):
    nvec = chunk // 16
    mesh = plsc.VectorSubcoreMesh(core_axis_name="c", subcore_axis_name="s")
    out_len = N * CAP + DUMP

    @functools.partial(
        pl.kernel, mesh=mesh,
        compiler_params=pltpu.CompilerParams(needs_layout_passes=False),
        out_type=[
            jax.ShapeDtypeStruct((out_len,), jnp.float32),
            jax.ShapeDtypeStruct((out_len,), jnp.int32),
        ],
        scratch_types=[
            pltpu.VMEM((chunk,), jnp.float32),
            pltpu.VMEM((16,), jnp.float32),
            pltpu.VMEM((16,), jnp.int32),
            pltpu.VMEM((CAP + 16,), jnp.float32),
            pltpu.VMEM((CAP + 16,), jnp.int32),
        ],
    )
    def sc_compact(scores_hbm, tau_hbm, base_hbm, outv_hbm, outi_hbm,
                   chunk_v, tau_v, base_v, valbuf, idxbuf):
        cid = lax.axis_index("c")
        sid = lax.axis_index("s")
        wid = sid * 2 + cid
        iota16 = lax.iota(jnp.int32, 16)
        one16 = jnp.ones((16,), jnp.int32)
        zero16 = jnp.zeros((16,), jnp.int32)

        for n in range(N):
            pltpu.sync_copy(
                scores_hbm.at[pl.ds(n * total + wid * chunk, chunk)], chunk_v)
            pltpu.sync_copy(tau_hbm.at[n], tau_v)
            pltpu.sync_copy(base_hbm.at[n, wid], base_v)
            tau16 = tau_v[...]
            base16 = base_v[...]

            def scatter_vec(i, ptrv):
                vals = chunk_v[pl.ds(i * 16, 16)]
                m = vals >= tau16
                mi = jnp.where(m, one16, zero16)
                incl = plsc.cumsum(mi)
                slots = ptrv + incl - mi
                ok = m & (slots < CAP + 16)
                fidx = wid * chunk + i * 16 + iota16
                plsc.store_scatter(valbuf, [slots], vals, mask=ok)
                plsc.store_scatter(idxbuf, [slots], fidx, mask=ok)
                return ptrv + plsc.all_reduce_population_count(m)

            def bbody(b, ptrv):
                i0 = b * 8
                mx = chunk_v[pl.ds(i0 * 16, 16)]
                for k in range(1, 8):
                    mx = jnp.maximum(mx, chunk_v[pl.ds((i0 + k) * 16, 16)])
                pc = plsc.all_reduce_population_count(mx >= tau16)

                def hitpath(p):
                    for k in range(8):
                        p = scatter_vec(i0 + k, p)
                    return p

                return lax.cond(pc[0] > 0, hitpath, lambda p: p, ptrv)

            cwv = lax.fori_loop(0, nvec // 8, bbody,
                                jnp.zeros((16,), jnp.int32))

            dump16 = jnp.int32(N * CAP) + wid * 16 + iota16

            def wbody(k, _):
                valid = (k * 16 + iota16) < cwv
                tgt = base16 + k * 16 + iota16
                safe = tgt < (n + 1) * CAP
                gidx = jnp.where(valid & safe, tgt, dump16)
                pltpu.sync_copy(valbuf.at[pl.ds(k * 16, 16)],
                                outv_hbm.at[gidx])
                pltpu.sync_copy(idxbuf.at[pl.ds(k * 16, 16)],
                                outi_hbm.at[gidx])
                return 0

            cw = cwv[0]
            lax.fori_loop(0, (cw + 15) >> 4, wbody, 0)

    return sc_compact


def kernel(locations, box_cls, box_regression, centerness, image_sizes):
    N, C, H, W = box_cls.shape
    HW = H * W
    total = HW * LANES
    chunk = total // NW
    cls_nat = box_cls.reshape(N, C, HW)
    ctr_nat = centerness.reshape(N, 1, HW)

    scores, tau, counts = _tc_select(cls_nat, ctr_nat)

    counts = counts.reshape(N, NW)
    offs = jnp.cumsum(counts, axis=1) - counts
    base = (jnp.arange(N, dtype=jnp.int32)[:, None] * CAP + offs)
    base16 = jnp.broadcast_to(base[:, :, None], (N, NW, 16)).astype(jnp.int32)
    tau_f = lax.bitcast_convert_type(tau.reshape(N, 1), jnp.float32)
    tau16 = jnp.broadcast_to(tau_f, (N, 16))
    c2 = counts.sum(axis=1)

    sc_compact = _make_sc_compact(N, total, chunk)
    outv_flat, outi_flat = sc_compact(scores.reshape(-1), tau16, base16)

    valid = jnp.arange(CAP, dtype=jnp.int32)[None, :] < c2[:, None]
    vals = jnp.where(valid, outv_flat[:N * CAP].reshape(N, CAP), -2.0)
    idxp = jnp.where(valid, outi_flat[:N * CAP].reshape(N, CAP), 0)
    idxl = (idxp >> 7) * C + (idxp & (LANES - 1))

    top_vals, tp = lax.top_k(vals, PRE_NMS_TOP_N)
    top_idx = jnp.take_along_axis(idxl, tp, axis=1)

    reg = jnp.transpose(box_regression, (0, 2, 3, 1)).reshape(N, HW, 4)
    loc_idx = top_idx // C
    labels = (top_idx % C) + 1
    per_reg = jnp.take_along_axis(reg, loc_idx[:, :, None], axis=1)
    per_loc = locations[loc_idx]
    x1 = per_loc[..., 0] - per_reg[..., 0]
    y1 = per_loc[..., 1] - per_reg[..., 1]
    x2 = per_loc[..., 0] + per_reg[..., 2]
    y2 = per_loc[..., 1] + per_reg[..., 3]
    w = jnp.maximum(image_sizes[:, 1], 2).astype(jnp.float32)[:, None]
    h = jnp.maximum(image_sizes[:, 0], 2).astype(jnp.float32)[:, None]
    x1 = jnp.clip(x1, 0.0, w - 1.0)
    x2 = jnp.clip(x2, 0.0, w - 1.0)
    y1 = jnp.clip(y1, 0.0, h - 1.0)
    y2 = jnp.clip(y2, 0.0, h - 1.0)
    ws = x2 - x1 + 1.0
    hs = y2 - y1 + 1.0
    keep = (ws >= 0) & (hs >= 0) & (top_vals > 0.0)
    final_scores = jnp.where(keep, top_vals, -1.0)
    fin_vals, fin_idx = lax.top_k(final_scores, FPN_POST_NMS_TOP_N)
    boxes = jnp.stack([x1, y1, x2, y2], axis=-1)
    fin_boxes = jnp.take_along_axis(boxes, fin_idx[:, :, None], axis=1)
    fin_labels = jnp.take_along_axis(labels, fin_idx, axis=1).astype(jnp.float32)
    out = jnp.concatenate([fin_boxes, fin_vals[:, :, None], fin_labels[:, :, None]], axis=-1)
    return out


# speedup vs baseline: 1.5545x; 1.0548x over previous
import functools

import jax
import jax.numpy as jnp
from jax import lax
from jax.experimental import pallas as pl
from jax.experimental.pallas import tpu as pltpu
from jax.experimental.pallas import tpu_sc as plsc

PRE_NMS_THRESH = 0.01
PRE_NMS_TOP_N = 1000
FPN_POST_NMS_TOP_N = 100

CAP = 2048
LANES = 128
SCORE_CHUNK = 1152
COUNT_CHUNK = 1296
NW = 32
HI_BITS = 0x3F800001
DUMP = 1024


def _score_body(cls_ref, ctr_ref, s_ref, tau_ref, cnt_ref, sub_ref):
    C, HW = cls_ref.shape[1], cls_ref.shape[2]
    pad = jnp.full((SCORE_CHUNK, LANES - 80), -1.0, jnp.float32)
    sub_rows = SCORE_CHUNK // 16

    def score_chunk(j, _):
        c = cls_ref[0, :, pl.ds(j * SCORE_CHUNK, SCORE_CHUNK)]
        t = ctr_ref[0, :, pl.ds(j * SCORE_CHUNK, SCORE_CHUNK)]
        sT = jnp.transpose(jax.nn.sigmoid(c))
        stT = jnp.transpose(jax.nn.sigmoid(t))
        msk = jnp.where(sT > PRE_NMS_THRESH, sT * stT, -1.0)
        full = jnp.concatenate([msk, pad], axis=1)
        s_ref[0, pl.ds(j * SCORE_CHUNK, SCORE_CHUNK), :] = full
        sub_ref[pl.ds(j * sub_rows, sub_rows), :] = full[0:sub_rows, :]
        return 0
    lax.fori_loop(0, HW // SCORE_CHUNK, score_chunk, 0)

    def count_ge(tau_bits):
        def cbody(i, acc):
            blk = s_ref[0, pl.ds(i * COUNT_CHUNK, COUNT_CHUNK), :]
            bits = lax.bitcast_convert_type(blk, jnp.int32)
            return acc + jnp.sum((bits >= tau_bits).astype(jnp.int32))
        return lax.fori_loop(0, HW // COUNT_CHUNK, cbody, jnp.int32(0))

    def count_sub(tau_bits):
        bits = lax.bitcast_convert_type(sub_ref[...], jnp.int32)
        return jnp.sum((bits >= tau_bits).astype(jnp.int32))

    SUB_LO, SUB_HI = 88, 126

    def sbis_cond(st):
        lo, hi, cnt, it = st
        bad = (cnt < SUB_LO) | (cnt > SUB_HI)
        return bad & (it < 24) & (lo + 1 < hi)

    def sbis_body(st):
        lo, hi, cnt, it = st
        mid = (lo + hi) // 2
        c = count_sub(mid)
        ok = c >= SUB_LO
        return (jnp.where(ok, mid, lo), jnp.where(ok, hi, mid),
                jnp.where(ok, c, cnt), it + 1)

    t1, _, _, _ = lax.while_loop(
        sbis_cond, sbis_body,
        (jnp.int32(0), jnp.int32(HI_BITS), count_sub(jnp.int32(0)),
         jnp.int32(0)))

    rows_w = HW // NW
    lane = lax.broadcasted_iota(jnp.int32, (1, NW), 1)

    def wcounts(tau_bits):
        def wcount(w, cvec):
            blk = s_ref[0, pl.ds(w * rows_w, rows_w), :]
            bits = lax.bitcast_convert_type(blk, jnp.int32)
            cw = jnp.sum((bits >= tau_bits).astype(jnp.int32))
            return jnp.where(lane == w, cw, cvec)
        return lax.fori_loop(0, NW, wcount, jnp.zeros((1, NW), jnp.int32))

    def bis_cond(st):
        lo, hi, cnt, it = st
        bad = (cnt < PRE_NMS_TOP_N) | (cnt > CAP)
        return bad & (it < 34) & (lo + 1 < hi)

    def bis_body(st):
        lo, hi, cnt, it = st
        mid = (lo + hi) // 2
        c = count_ge(mid)
        ok = c >= PRE_NMS_TOP_N
        return (jnp.where(ok, mid, lo), jnp.where(ok, hi, mid),
                jnp.where(ok, c, cnt), it + 1)

    cvec1 = wcounts(t1)
    c1 = jnp.sum(cvec1)
    lo0 = jnp.where(c1 >= PRE_NMS_TOP_N, t1, jnp.int32(0))
    hi0 = jnp.where(c1 >= PRE_NMS_TOP_N, jnp.int32(HI_BITS), t1)
    cnt0 = jnp.where(c1 >= PRE_NMS_TOP_N, c1, jnp.int32(0x7FFFFFF0))
    tau, _, _, _ = lax.while_loop(
        bis_cond, bis_body, (lo0, hi0, cnt0, jnp.int32(0)))
    tau_ref[...] = lax.broadcast(tau, (1, 1, 1))

    cvec = lax.cond(tau == t1, lambda: cvec1, lambda: wcounts(tau))
    cnt_ref[...] = cvec.reshape(1, 1, NW)


def _tc_select(cls_nat, ctr_nat):
    N, C, HW = cls_nat.shape
    return pl.pallas_call(
        _score_body,
        grid=(N,),
        in_specs=[
            pl.BlockSpec((1, C, HW), lambda n: (n, 0, 0)),
            pl.BlockSpec((1, 1, HW), lambda n: (n, 0, 0)),
        ],
        out_specs=[
            pl.BlockSpec((1, HW, LANES), lambda n: (n, 0, 0)),
            pl.BlockSpec((1, 1, 1), lambda n: (n, 0, 0)),
            pl.BlockSpec((1, 1, NW), lambda n: (n, 0, 0)),
        ],
        out_shape=[
            jax.ShapeDtypeStruct((N, HW, LANES), jnp.float32),
            jax.ShapeDtypeStruct((N, 1, 1), jnp.int32),
            jax.ShapeDtypeStruct((N, 1, NW), jnp.int32),
        ],
        scratch_shapes=[pltpu.VMEM((HW // 16, LANES), jnp.float32)],
    )(cls_nat, ctr_nat)


def _make_sc_compact(N, total, chunk):
    nvec = chunk // 16
    mesh = plsc.VectorSubcoreMesh(core_axis_name="c", subcore_axis_name="s")
    out_len = N * CAP + DUMP

    @functools.partial(
        pl.kernel, mesh=mesh,
        compiler_params=pltpu.CompilerParams(needs_layout_passes=False),
        out_type=[
            jax.ShapeDtypeStruct((out_len,), jnp.float32),
            jax.ShapeDtypeStruct((out_len,), jnp.int32),
        ],
        scratch_types=[
            pltpu.VMEM((chunk // 2,), jnp.float32),
            pltpu.VMEM((chunk // 2,), jnp.float32),
            pltpu.VMEM((16,), jnp.float32),
            pltpu.VMEM((16,), jnp.int32),
            pltpu.VMEM((CAP + 16,), jnp.float32),
            pltpu.VMEM((CAP + 16,), jnp.int32),
            pltpu.SemaphoreType.DMA,
            pltpu.SemaphoreType.DMA,
        ],
    )
    def sc_compact(scores_hbm, tau_hbm, base_hbm, outv_hbm, outi_hbm,
                   buf_a, buf_b, tau_v, base_v, valbuf, idxbuf,
                   sem_a, sem_b):
        cid = lax.axis_index("c")
        sid = lax.axis_index("s")
        wid = sid * 2 + cid
        iota16 = lax.iota(jnp.int32, 16)
        one16 = jnp.ones((16,), jnp.int32)
        zero16 = jnp.zeros((16,), jnp.int32)
        half = chunk // 2
        bufs = (buf_a, buf_b), (sem_a, sem_b)

        def start(t):
            n, h = divmod(t, 2)
            buf, sem = bufs[0][t % 2], bufs[1][t % 2]
            return pltpu.async_copy(
                scores_hbm.at[pl.ds(n * total + wid * chunk + h * half,
                                    half)],
                buf, sem)

        pending = start(0)
        for t in range(2 * N):
            n, h = divmod(t, 2)
            buf = bufs[0][t % 2]
            if h == 0:
                pltpu.sync_copy(tau_hbm.at[n], tau_v)
                pltpu.sync_copy(base_hbm.at[n, wid], base_v)
            tau16 = tau_v[...]
            base16 = base_v[...]
            pending.wait()
            if t + 1 < 2 * N:
                pending = start(t + 1)

            def scatter_vec(i, ptrv, buf=buf, tau16=tau16, h=h):
                vals = buf[pl.ds(i * 16, 16)]
                m = vals >= tau16
                mi = jnp.where(m, one16, zero16)
                incl = plsc.cumsum(mi)
                slots = ptrv + incl - mi
                ok = m & (slots < CAP + 16)
                fidx = wid * chunk + h * half + i * 16 + iota16
                plsc.store_scatter(valbuf, [slots], vals, mask=ok)
                plsc.store_scatter(idxbuf, [slots], fidx, mask=ok)
                return ptrv + plsc.all_reduce_population_count(m)

            def bbody(b, ptrv, buf=buf, tau16=tau16, sv=scatter_vec):
                i0 = b * 8
                mx = buf[pl.ds(i0 * 16, 16)]
                for k in range(1, 8):
                    mx = jnp.maximum(mx, buf[pl.ds((i0 + k) * 16, 16)])
                pc = plsc.all_reduce_population_count(mx >= tau16)

                def hitpath(p):
                    for k in range(8):
                        p = sv(i0 + k, p)
                    return p

                return lax.cond(pc[0] > 0, hitpath, lambda p: p, ptrv)

            ptr0 = (jnp.zeros((16,), jnp.int32) if h == 0 else cwv)
            cwv = lax.fori_loop(0, half // 128, bbody, ptr0)
            if h == 0:
                continue

            dump16 = jnp.int32(N * CAP) + wid * 16 + iota16

            def wbody(k, _):
                valid = (k * 16 + iota16) < cwv
                tgt = base16 + k * 16 + iota16
                safe = tgt < (n + 1) * CAP
                gidx = jnp.where(valid & safe, tgt, dump16)
                pltpu.sync_copy(valbuf.at[pl.ds(k * 16, 16)],
                                outv_hbm.at[gidx])
                pltpu.sync_copy(idxbuf.at[pl.ds(k * 16, 16)],
                                outi_hbm.at[gidx])
                return 0

            cw = cwv[0]
            lax.fori_loop(0, (cw + 15) >> 4, wbody, 0)

    return sc_compact


def kernel(locations, box_cls, box_regression, centerness, image_sizes):
    N, C, H, W = box_cls.shape
    HW = H * W
    total = HW * LANES
    chunk = total // NW
    cls_nat = box_cls.reshape(N, C, HW)
    ctr_nat = centerness.reshape(N, 1, HW)

    scores, tau, counts = _tc_select(cls_nat, ctr_nat)

    counts = counts.reshape(N, NW)
    offs = jnp.cumsum(counts, axis=1) - counts
    base = (jnp.arange(N, dtype=jnp.int32)[:, None] * CAP + offs)
    base16 = jnp.broadcast_to(base[:, :, None], (N, NW, 16)).astype(jnp.int32)
    tau_f = lax.bitcast_convert_type(tau.reshape(N, 1), jnp.float32)
    tau16 = jnp.broadcast_to(tau_f, (N, 16))
    c2 = counts.sum(axis=1)

    sc_compact = _make_sc_compact(N, total, chunk)
    outv_flat, outi_flat = sc_compact(scores.reshape(-1), tau16, base16)

    valid = jnp.arange(CAP, dtype=jnp.int32)[None, :] < c2[:, None]
    vals = jnp.where(valid, outv_flat[:N * CAP].reshape(N, CAP), -2.0)
    idxp = jnp.where(valid, outi_flat[:N * CAP].reshape(N, CAP), 0)
    idxl = (idxp >> 7) * C + (idxp & (LANES - 1))

    top_vals, tp = lax.top_k(vals, PRE_NMS_TOP_N)
    top_idx = jnp.take_along_axis(idxl, tp, axis=1)

    reg = jnp.transpose(box_regression, (0, 2, 3, 1)).reshape(N, HW, 4)
    loc_idx = top_idx // C
    labels = (top_idx % C) + 1
    per_reg = jnp.take_along_axis(reg, loc_idx[:, :, None], axis=1)
    per_loc = locations[loc_idx]
    x1 = per_loc[..., 0] - per_reg[..., 0]
    y1 = per_loc[..., 1] - per_reg[..., 1]
    x2 = per_loc[..., 0] + per_reg[..., 2]
    y2 = per_loc[..., 1] + per_reg[..., 3]
    w = jnp.maximum(image_sizes[:, 1], 2).astype(jnp.float32)[:, None]
    h = jnp.maximum(image_sizes[:, 0], 2).astype(jnp.float32)[:, None]
    x1 = jnp.clip(x1, 0.0, w - 1.0)
    x2 = jnp.clip(x2, 0.0, w - 1.0)
    y1 = jnp.clip(y1, 0.0, h - 1.0)
    y2 = jnp.clip(y2, 0.0, h - 1.0)
    ws = x2 - x1 + 1.0
    hs = y2 - y1 + 1.0
    keep = (ws >= 0) & (hs >= 0) & (top_vals > 0.0)
    final_scores = jnp.where(keep, top_vals, -1.0)
    fin_vals, fin_idx = lax.top_k(final_scores, FPN_POST_NMS_TOP_N)
    boxes = jnp.stack([x1, y1, x2, y2], axis=-1)
    fin_boxes = jnp.take_along_axis(boxes, fin_idx[:, :, None], axis=1)
    fin_labels = jnp.take_along_axis(labels, fin_idx, axis=1).astype(jnp.float32)
    out = jnp.concatenate([fin_boxes, fin_vals[:, :, None], fin_labels[:, :, None]], axis=-1)
    return out
